# Initial kernel scaffold; baseline (speedup 1.0000x reference)
import jax
import jax.numpy as jnp
from jax.experimental import pallas as pl


def kernel(focus_table, context_table, focus_idx, context_idx, neg_context_idx):
    raise NotImplementedError("write your pallas kernel here")


# speedup vs baseline: 5.1959x; 5.1959x over previous
import functools

import jax
import jax.numpy as jnp
from jax import lax
from jax.experimental import pallas as pl
from jax.experimental.pallas import tpu as pltpu
from jax.experimental.pallas import tpu_sc as plsc

B = 16384
D = 64
K = 20

NC = 2
NS = 16
NW = NC * NS
BW = B // NW
GB = 4
GROUPS = BW // GB
GROW = GB * K
NCHUNK = 128
NFC = BW // NCHUNK


def _sc_scores(focus_table, context_table, fi2, ci2, ni2):
    mesh = plsc.VectorSubcoreMesh(
        core_axis_name="c", subcore_axis_name="s", num_cores=NC,
        num_subcores=NS)

    @functools.partial(
        pl.kernel,
        out_type=(
            jax.ShapeDtypeStruct((B, 16), jnp.float32),
            jax.ShapeDtypeStruct((B, 16), jnp.float32),
        ),
        mesh=mesh,
        compiler_params=pltpu.CompilerParams(use_tc_tiling_on_sc=False),
        scratch_types=[
            pltpu.VMEM((NFC, NCHUNK), jnp.int32),
            pltpu.VMEM((NFC, NCHUNK), jnp.int32),
            pltpu.VMEM((GROUPS, GROW), jnp.int32),
            pltpu.VMEM((BW, D), jnp.float32),
            pltpu.VMEM((BW, D), jnp.float32),
            pltpu.VMEM((2, GROW, D), jnp.float32),
            pltpu.VMEM((BW, 16), jnp.float32),
            pltpu.VMEM((BW, 16), jnp.float32),
            pltpu.SemaphoreType.DMA,
            pltpu.SemaphoreType.DMA,
        ],
    )
    def k(ft_hbm, ct_hbm, fi_hbm, ci_hbm, ni_hbm, posi_hbm, neg_hbm,
          fidx_v, cidx_v, nidx_v, fe_v, ce_v, nce_v, posi_v, neg_v,
          sem_fc, sem_n):
        wid = lax.axis_index("s") * NC + lax.axis_index("c")
        base = wid * BW

        pltpu.sync_copy(fi_hbm.at[pl.ds(wid * NFC, NFC)], fidx_v)
        pltpu.sync_copy(ci_hbm.at[pl.ds(wid * NFC, NFC)], cidx_v)
        pltpu.sync_copy(ni_hbm.at[pl.ds(wid * GROUPS, GROUPS)], nidx_v)

        for j in range(NFC):
            pltpu.make_async_copy(
                ft_hbm.at[fidx_v.at[j]],
                fe_v.at[pl.ds(j * NCHUNK, NCHUNK)], sem_fc).start()
        for j in range(NFC):
            pltpu.make_async_copy(
                ct_hbm.at[cidx_v.at[j]],
                ce_v.at[pl.ds(j * NCHUNK, NCHUNK)], sem_fc).start()
        pltpu.make_async_copy(
            ct_hbm.at[nidx_v.at[0]], nce_v.at[0], sem_n).start()
        for j in range(NFC):
            pltpu.make_async_copy(
                ft_hbm.at[fidx_v.at[j]],
                fe_v.at[pl.ds(j * NCHUNK, NCHUNK)], sem_fc).wait()
            pltpu.make_async_copy(
                ct_hbm.at[cidx_v.at[j]],
                ce_v.at[pl.ds(j * NCHUNK, NCHUNK)], sem_fc).wait()

        def group_body(g, carry):
            par = lax.rem(g, 2)
            pltpu.make_async_copy(
                ct_hbm.at[nidx_v.at[g]], nce_v.at[par], sem_n).wait()

            @pl.when(g < GROUPS - 1)
            def _():
                pltpu.make_async_copy(
                    ct_hbm.at[nidx_v.at[g + 1]],
                    nce_v.at[1 - par], sem_n).start()

            for bb in range(GB):
                b = g * GB + bb
                f = [fe_v[b, pl.ds(j * 16, 16)] for j in range(4)]
                acc = [jnp.zeros((16,), jnp.float32) for _ in range(4)]
                for kk in range(K):
                    r = bb * K + kk
                    for j in range(4):
                        acc[j] = acc[j] + nce_v[par, r, pl.ds(j * 16, 16)] * f[j]
                neg_v[b, :] = acc[0] + acc[1] + acc[2] + acc[3]
                c = [ce_v[b, pl.ds(j * 16, 16)] for j in range(4)]
                posi_v[b, :] = (
                    c[0] * f[0] + c[1] * f[1] + c[2] * f[2] + c[3] * f[3])
            return carry

        lax.fori_loop(0, GROUPS, group_body, 0)

        pltpu.sync_copy(posi_v, posi_hbm.at[pl.ds(base, BW)])
        pltpu.sync_copy(neg_v, neg_hbm.at[pl.ds(base, BW)])

    return k(focus_table, context_table, fi2, ci2, ni2)


def _tc_loss_body(p_ref, n_ref, o_ref):
    i = lax.broadcasted_iota(jnp.int32, (128, 8), 0)
    j = lax.broadcasted_iota(jnp.int32, (128, 8), 1)
    m = jnp.where(i // 16 == j, 1.0, 0.0).astype(jnp.float32)
    dn = (((1,), (0,)), ((), ()))
    ps = lax.dot_general(p_ref[...], m, dn, precision=lax.Precision.HIGHEST)
    ns = lax.dot_general(n_ref[...], m, dn, precision=lax.Precision.HIGHEST)
    ls_p = jnp.minimum(ps, 0.0) - jnp.log1p(jnp.exp(-jnp.abs(ps)))
    ls_n = jnp.minimum(ns, 0.0) - jnp.log1p(jnp.exp(-jnp.abs(ns)))
    o_ref[0, 0] = jnp.sum(jnp.square(1.0 - ls_p)) + jnp.sum(jnp.square(ls_n))


def _tc_loss(posi_part, neg_part):
    out = pl.pallas_call(
        _tc_loss_body,
        out_shape=jax.ShapeDtypeStruct((1, 1), jnp.float32),
        in_specs=[
            pl.BlockSpec(memory_space=pltpu.VMEM),
            pl.BlockSpec(memory_space=pltpu.VMEM),
        ],
        out_specs=pl.BlockSpec(memory_space=pltpu.SMEM),
    )(posi_part.reshape(B // 8, 128), neg_part.reshape(B // 8, 128))
    return out.reshape(())


def kernel(focus_table, context_table, focus_idx, context_idx,
           neg_context_idx):
    fi2 = focus_idx.astype(jnp.int32).reshape(B // NCHUNK, NCHUNK)
    ci2 = context_idx.astype(jnp.int32).reshape(B // NCHUNK, NCHUNK)
    ni2 = neg_context_idx.astype(jnp.int32).reshape(B * K // GROW, GROW)
    posi, neg = _sc_scores(focus_table, context_table, fi2, ci2, ni2)
    return _tc_loss(posi, neg)
